# Initial kernel scaffold; baseline (speedup 1.0000x reference)
import jax
import jax.numpy as jnp
from jax.experimental import pallas as pl


def kernel(x, edge_index, batch, W1, b1, Wg, bg):
    raise NotImplementedError("write your pallas kernel here")


# speedup vs baseline: 70.8157x; 70.8157x over previous
import jax
import jax.numpy as jnp
from jax import lax
from jax.experimental import pallas as pl
from jax.experimental.pallas import tpu as pltpu
from jax.experimental.pallas import tpu_sc as plsc

N = 10000
E = 320000
D = 128
H = 32
G = 64

NC = 2
NS = 16
NP = 10240
CH = NP // NS
CH32 = NP // (NC * NS)
KB = 128
EC = E // NC
ETT = 9984
EXD = (E - 32 * ETT) // KB
EXM = (EC - NS * ETT) // KB
RB = 768
NBIG = ETT // RB
NBUF = 3

_MESH = plsc.VectorSubcoreMesh(core_axis_name="c", subcore_axis_name="s")
_CPARAMS = pltpu.CompilerParams(
    needs_layout_passes=False, use_tc_tiling_on_sc=False)
_NEG = -3.4028235e38


def _zero_ref(ref, n, value=0.0):
    v = jnp.full((16,), value, jnp.float32)

    def body(i, _):
        ref[pl.ds(i * 16, 16)] = v
        return _

    lax.fori_loop(0, n // 16, body, 0)


_DEG_KW = dict(
    compiler_params=_CPARAMS,
    out_type=jax.ShapeDtypeStruct((NC * NP,), jnp.float32),
    mesh=_MESH,
    scratch_types=[
        pltpu.VMEM((NP,), jnp.float32),
        pltpu.VMEM((ETT + KB,), jnp.int32),
        pltpu.VMEM((CH,), jnp.float32),
        pltpu.VMEM((NS * CH,), jnp.float32),
        pltpu.SemaphoreType.DMA,
        pltpu.VMEM_SHARED((NS * NP,), jnp.float32),
    ],
)


def _deg_body(dst, degp, deg_loc, stage, dacc, dbig, sem, slots):
    c = lax.axis_index("c")
    s = lax.axis_index("s")
    w = c * NS + s
    d0 = pltpu.async_copy(dst.at[pl.ds(w * ETT, ETT)],
                          stage.at[pl.ds(0, ETT)], sem)
    _zero_ref(deg_loc, NP)
    d0.wait()
    ones = jnp.ones((16,), jnp.float32)

    def scat(lo, n):
        def inner(i, _2):
            idx = stage[pl.ds(lo + i * 16, 16)]
            plsc.addupdate_scatter(deg_loc, [idx], ones)
            return _2

        lax.fori_loop(0, n // 16, inner, 0)

    scat(0, ETT)

    @pl.when(w < EXD)
    def _extra():
        pltpu.sync_copy(dst.at[pl.ds(32 * ETT + w * KB, KB)],
                        stage.at[pl.ds(ETT, KB)])
        scat(ETT, KB)

    pltpu.sync_copy(deg_loc, slots.at[pl.ds(s * NP, NP)])
    plsc.subcore_barrier()

    noff = s * CH
    descs = [
        pltpu.async_copy(slots.at[pl.ds(j * NP + noff, CH)],
                         dbig.at[pl.ds(j * CH, CH)], sem)
        for j in range(NS)
    ]
    for d in descs:
        d.wait()

    def red(i, _):
        b = i * 16
        v = dbig[pl.ds(b, 16)]
        for j in range(1, NS):
            v = v + dbig[pl.ds(j * CH + b, 16)]
        dacc[pl.ds(b, 16)] = v
        return _

    lax.fori_loop(0, CH // 16, red, 0)
    pltpu.sync_copy(dacc, degp.at[pl.ds(c * NP + noff, CH)])


_deg = pl.kernel(_deg_body, **_DEG_KW)


def _scale_body(x_ref, w_ref, d_ref, y_ref):
    xw = jnp.dot(x_ref[...], w_ref[...], preferred_element_type=jnp.float32)
    d = d_ref[...]
    dinv = lax.rsqrt(1.0 + d[0] + d[1])
    y_ref[...] = xw * dinv[:, None]


def _scale(x_pad, W1, degp):
    blk = 1024
    return pl.pallas_call(
        _scale_body,
        grid=(NP // blk,),
        in_specs=[
            pl.BlockSpec((blk, D), lambda i: (i, 0)),
            pl.BlockSpec((D, H), lambda i: (0, 0)),
            pl.BlockSpec((NC, blk), lambda i: (0, i)),
        ],
        out_specs=pl.BlockSpec((blk, H), lambda i: (i, 0)),
        out_shape=jax.ShapeDtypeStruct((NP, H), jnp.float32),
    )(x_pad, W1, degp)


_MSG_KW = dict(
    compiler_params=_CPARAMS,
    out_type=jax.ShapeDtypeStruct((NC * NP, H), jnp.float32),
    mesh=_MESH,
    scratch_types=[
        pltpu.VMEM((ETT + KB,), jnp.int32),
        pltpu.VMEM((ETT + KB,), jnp.int32),
        pltpu.VMEM((RB, H), jnp.float32),
        pltpu.VMEM((RB, H), jnp.float32),
        pltpu.VMEM((RB, H), jnp.float32),
        pltpu.SemaphoreType.DMA,
        pltpu.SemaphoreType.DMA,
        pltpu.SemaphoreType.DMA,
        pltpu.SemaphoreType.DMA,
        pltpu.SemaphoreType.DMA,
        pltpu.SemaphoreType.DMA,
        pltpu.VMEM_SHARED((NP, H), jnp.float32),
    ],
)


def _msg_body(src, dst, y, zeros, accp, sidx, didx, r0, r1, r2,
              g0, g1, g2, s0, s1, s2, acc_sh):
    c = lax.axis_index("c")
    s = lax.axis_index("s")
    noff = s * CH
    rows = [r0, r1, r2]
    gsem = [g0, g1, g2]
    ssem = [s0, s1, s2]
    dz = pltpu.async_copy(zeros.at[pl.ds(noff, CH)],
                          acc_sh.at[pl.ds(noff, CH)], g0)
    ebase = c * EC + s * ETT
    ds1 = pltpu.async_copy(src.at[pl.ds(ebase, ETT)],
                           sidx.at[pl.ds(0, ETT)], g1)
    ds2 = pltpu.async_copy(dst.at[pl.ds(ebase, ETT)],
                           didx.at[pl.ds(0, ETT)], g2)

    @pl.when(s < EXM)
    def _extra_stage():
        xb = c * EC + NS * ETT + s * KB
        pltpu.sync_copy(src.at[pl.ds(xb, KB)], sidx.at[pl.ds(ETT, KB)])
        pltpu.sync_copy(dst.at[pl.ds(xb, KB)], didx.at[pl.ds(ETT, KB)])

    dz.wait()
    ds1.wait()
    ds2.wait()
    plsc.subcore_barrier()

    def g_start(j):
        b = j % NBUF
        ids = sidx.at[pl.ds(j * RB, RB)]
        return pltpu.async_copy(y.at[ids], rows[b], gsem[b])

    def s_start(j):
        b = j % NBUF
        idd = didx.at[pl.ds(j * RB, RB)]
        return pltpu.async_copy(rows[b], acc_sh.at[idd], ssem[b], add=True)

    gd = [None] * NBIG
    sd = [None] * NBIG
    for j in range(min(NBUF - 1, NBIG)):
        gd[j] = g_start(j)
    for j in range(NBIG):
        nx = j + NBUF - 1
        if nx < NBIG:
            if nx >= NBUF:
                sd[nx - NBUF].wait()
            gd[nx] = g_start(nx)
        gd[j].wait()
        sd[j] = s_start(j)
    for j in range(max(0, NBIG - NBUF), NBIG):
        sd[j].wait()

    @pl.when(s < EXM)
    def _extra_blk():
        ids = sidx.at[pl.ds(ETT, KB)]
        idd = didx.at[pl.ds(ETT, KB)]
        rr = r0.at[pl.ds(0, KB)]
        pltpu.async_copy(y.at[ids], rr, g0).wait()
        pltpu.sync_copy(rr, acc_sh.at[idd], add=True)

    plsc.subcore_barrier()
    pltpu.sync_copy(acc_sh.at[pl.ds(noff, CH)],
                    accp.at[pl.ds(c * NP + noff, CH)])


_msg = pl.kernel(_msg_body, **_MSG_KW)


_TBL = (G + 1) * H
_SL = 2176
CHP = 384
_R = G * H // NS

_POOL_KW = dict(
    compiler_params=_CPARAMS,
    out_type=(
        jax.ShapeDtypeStruct((NC * G * H,), jnp.float32),
        jax.ShapeDtypeStruct((NC * G * H,), jnp.float32),
    ),
    mesh=_MESH,
    scratch_types=[
        pltpu.VMEM((CH32 * H,), jnp.float32),
        pltpu.VMEM((CH32 * H,), jnp.float32),
        pltpu.VMEM((CH32 * H,), jnp.float32),
        pltpu.VMEM((CHP,), jnp.float32),
        pltpu.VMEM((CHP,), jnp.float32),
        pltpu.VMEM((CHP,), jnp.int32),
        pltpu.VMEM((H,), jnp.float32),
        pltpu.VMEM((_TBL,), jnp.float32),
        pltpu.VMEM((_TBL,), jnp.float32),
        pltpu.VMEM((_R,), jnp.float32),
        pltpu.VMEM((_R,), jnp.float32),
        pltpu.VMEM((2 * NS * _R,), jnp.float32),
        pltpu.SemaphoreType.DMA,
        pltpu.VMEM_SHARED((NS * 2 * _SL,), jnp.float32),
    ],
)


def _pool_body(accf, yf, degp, batch, b1, sums, maxs,
               a0b, a1b, yb, d0b, dvb, bb, b1b, sumtab, maxtab, sacc, macc,
               rbig, sem, slots):
    c = lax.axis_index("c")
    s = lax.axis_index("s")
    w = c * NS + s
    foff = w * CH32 * H
    stg = [
        pltpu.async_copy(accf.at[pl.ds(foff, CH32 * H)], a0b, sem),
        pltpu.async_copy(accf.at[pl.ds(NP * H + foff, CH32 * H)], a1b, sem),
        pltpu.async_copy(yf.at[pl.ds(foff, CH32 * H)], yb, sem),
        pltpu.async_copy(degp.at[pl.ds(w * CHP, CHP)], d0b, sem),
        pltpu.async_copy(degp.at[pl.ds(32 * CHP + w * CHP, CHP)], dvb, sem),
        pltpu.async_copy(batch.at[pl.ds(w * CHP, CHP)], bb, sem),
        pltpu.async_copy(b1, b1b, sem),
    ]
    _zero_ref(sumtab, _TBL)
    _zero_ref(maxtab, _TBL, _NEG)
    for d in stg:
        d.wait()

    half = jnp.full((16,), 0.5, jnp.float32)
    th = jnp.full((16,), 1.5, jnp.float32)
    magic = jnp.full((16,), 0x5F3759DF, jnp.int32)

    def newton(i, _):
        dg = 1.0 + d0b[pl.ds(i * 16, 16)] + dvb[pl.ds(i * 16, 16)]
        ii = magic - (plsc.bitcast(dg, jnp.int32) >> 1)
        yv = plsc.bitcast(ii, jnp.float32)
        yv = yv * (th - half * dg * yv * yv)
        yv = yv * (th - half * dg * yv * yv)
        yv = yv * (th - half * dg * yv * yv)
        dvb[pl.ds(i * 16, 16)] = yv
        return _

    lax.fori_loop(0, CH32 // 16, newton, 0)

    b1lo = b1b[pl.ds(0, 16)]
    b1hi = b1b[pl.ds(16, 16)]
    iota = lax.broadcasted_iota(jnp.int32, (16,), 0)
    zero16 = jnp.zeros((16,), jnp.float32)
    zi = jnp.zeros((16,), jnp.int32)

    def node(n, _):
        gspl = plsc.load_gather(bb, [zi + n])
        dspl = plsc.load_gather(dvb, [zi + n])
        base = n * H
        lo = pl.ds(base, 16)
        hi = pl.ds(base + 16, 16)
        hlo = jnp.maximum(dspl * (a0b[lo] + a1b[lo] + yb[lo]) + b1lo, zero16)
        hhi = jnp.maximum(dspl * (a0b[hi] + a1b[hi] + yb[hi]) + b1hi, zero16)
        ilo = gspl * H + iota
        ihi = ilo + 16
        plsc.addupdate_scatter(sumtab, [ilo], hlo)
        plsc.addupdate_scatter(sumtab, [ihi], hhi)
        plsc.store_scatter(
            maxtab, [ilo], jnp.maximum(plsc.load_gather(maxtab, [ilo]), hlo))
        plsc.store_scatter(
            maxtab, [ihi], jnp.maximum(plsc.load_gather(maxtab, [ihi]), hhi))
        return _

    lax.fori_loop(0, CH32, node, 0)

    pltpu.sync_copy(sumtab, slots.at[pl.ds(s * 2 * _SL, _TBL)])
    pltpu.sync_copy(maxtab, slots.at[pl.ds(s * 2 * _SL + _SL, _TBL)])
    plsc.subcore_barrier()

    roff = s * _R
    descs = []
    for j in range(NS):
        descs.append(pltpu.async_copy(
            slots.at[pl.ds(j * 2 * _SL + roff, _R)],
            rbig.at[pl.ds(j * _R, _R)], sem))
        descs.append(pltpu.async_copy(
            slots.at[pl.ds(j * 2 * _SL + _SL + roff, _R)],
            rbig.at[pl.ds(NS * _R + j * _R, _R)], sem))
    for d in descs:
        d.wait()

    def red(i, _):
        b = i * 16
        sv = rbig[pl.ds(b, 16)]
        mv = rbig[pl.ds(NS * _R + b, 16)]
        for j in range(1, NS):
            sv = sv + rbig[pl.ds(j * _R + b, 16)]
            mv = jnp.maximum(mv, rbig[pl.ds(NS * _R + j * _R + b, 16)])
        sacc[pl.ds(b, 16)] = sv
        macc[pl.ds(b, 16)] = mv
        return _

    lax.fori_loop(0, _R // 16, red, 0)
    pltpu.sync_copy(sacc, sums.at[pl.ds(c * G * H + roff, _R)])
    pltpu.sync_copy(macc, maxs.at[pl.ds(c * G * H + roff, _R)])


_pool = pl.kernel(_pool_body, **_POOL_KW)


def _final_body(s_ref, m_ref, b_ref, wg_ref, bg_ref, o_ref):
    sp = s_ref[...]
    s2 = sp[0] + sp[1]
    mp = m_ref[...]
    mx = jnp.maximum(mp[0], mp[1])
    b = b_ref[...]
    ids = lax.broadcasted_iota(jnp.int32, (G, N), 0)
    cnt = jnp.sum(jnp.where(b == ids, 1.0, 0.0), axis=1)
    mean = s2 / jnp.maximum(cnt, 1.0)[:, None]
    feats = jnp.concatenate([s2, mean, mx], axis=1)
    o_ref[...] = (
        jnp.dot(feats, wg_ref[...], preferred_element_type=jnp.float32)
        + bg_ref[...]
    )


def _final(sums, maxs, batch2d, Wg, bg):
    return pl.pallas_call(
        _final_body,
        out_shape=jax.ShapeDtypeStruct((G, 1), jnp.float32),
    )(sums, maxs, batch2d, Wg, bg)


def kernel(x, edge_index, batch, W1, b1, Wg, bg):
    x_pad = jnp.pad(x, ((0, NP - N), (0, 0)))
    batch_pad = jnp.pad(batch, (0, NP - N), constant_values=G)
    src = edge_index[0]
    dst = edge_index[1]
    degp = _deg(dst)
    y = _scale(x_pad, W1, degp.reshape(NC, NP))
    zeros = jnp.zeros((NP, H), jnp.float32)
    accp = _msg(src, dst, y, zeros)
    degp_pad = jnp.pad(
        degp.reshape(NC * 32, CH32), ((0, 0), (0, CHP - CH32))).reshape(-1)
    batch_pad2 = jnp.pad(
        batch_pad.reshape(32, CH32), ((0, 0), (0, CHP - CH32)),
        constant_values=G).reshape(-1)
    sums, maxs = _pool(
        accp.reshape(NC * NP * H), y.reshape(NP * H), degp_pad, batch_pad2, b1)
    out = _final(sums.reshape(NC, G, H), maxs.reshape(NC, G, H),
                 batch.reshape(1, N), Wg, bg)
    return out
